# baseline (speedup 1.0000x reference)
import functools

import jax
import jax.numpy as jnp
from jax import lax
from jax.experimental import pallas as pl
from jax.experimental.pallas import tpu as pltpu
from jax.experimental.pallas import tpu_sc as plsc

NUM_GAUSSIANS = 1000000
D = 48
BATCH = 16384

_info = plsc.get_sparse_core_info()
_NC, _NS = _info.num_cores, _info.num_subcores
_NW = _NC * _NS
_B_PER_W = BATCH // _NW
_CHUNK = 128
_NCHUNK = _B_PER_W // _CHUNK


@jax.jit
def _gather(indices_grp, table_flat):
    mesh = plsc.VectorSubcoreMesh(core_axis_name="c", subcore_axis_name="s")

    @functools.partial(
        pl.kernel,
        out_type=jax.ShapeDtypeStruct((BATCH, D), jnp.float32),
        mesh=mesh,
        scratch_types=[
            pltpu.VMEM((_NCHUNK, _CHUNK), jnp.int32),
            pltpu.VMEM((_B_PER_W, D), jnp.float32),
            pltpu.SemaphoreType.DMA,
        ],
        compiler_params=pltpu.CompilerParams(use_tc_tiling_on_sc=False),
    )
    def body(idx_hbm, table_hbm, out_hbm, idx_v, rows_v, sem):
        wid = lax.axis_index("s") * _NC + lax.axis_index("c")
        pltpu.sync_copy(idx_hbm.at[wid], idx_v)
        copies = []
        for j in range(_NCHUNK):
            cp = pltpu.make_async_copy(
                table_hbm.at[idx_v.at[j]],
                rows_v.at[pl.ds(j * _CHUNK, _CHUNK)],
                sem,
            )
            cp.start()
            copies.append(cp)
        for cp in copies:
            cp.wait()
        pltpu.sync_copy(rows_v, out_hbm.at[pl.ds(wid * _B_PER_W, _B_PER_W)])

    return body(indices_grp, table_flat)


def kernel(indices, sh_coeffs):
    idx = jnp.asarray(indices, jnp.int32).reshape(_NW, _NCHUNK, _CHUNK)
    table_flat = sh_coeffs.reshape(NUM_GAUSSIANS, D)
    out = _gather(idx, table_flat)
    return out.reshape(BATCH, 16, 3)


# speedup vs baseline: 9.5584x; 9.5584x over previous
import functools

import jax
import jax.numpy as jnp
from jax import lax
from jax.experimental import pallas as pl
from jax.experimental.pallas import tpu as pltpu
from jax.experimental.pallas import tpu_sc as plsc

NUM_GAUSSIANS = 1000000
D = 48
BATCH = 16384
LANES = 128

_info = plsc.get_sparse_core_info()
_NC, _NS = _info.num_cores, _info.num_subcores
_NW = _NC * _NS
_B_PER_W = BATCH // _NW
_NBUF = 16
_NGRP = _B_PER_W // _NBUF


@jax.jit
def _gather(indices, table_t):
    mesh = plsc.VectorSubcoreMesh(core_axis_name="c", subcore_axis_name="s")

    @functools.partial(
        pl.kernel,
        out_type=jax.ShapeDtypeStruct((D, BATCH), jnp.float32),
        mesh=mesh,
        scratch_types=[
            pltpu.VMEM((_B_PER_W,), jnp.int32),
            pltpu.VMEM((_NBUF, D, LANES), jnp.float32),
            pltpu.VMEM((D, _B_PER_W), jnp.float32),
            pltpu.SemaphoreType.DMA,
            pltpu.SemaphoreType.DMA,
        ],
        compiler_params=pltpu.CompilerParams(needs_layout_passes=False),
    )
    def body(idx_hbm, table_hbm, out_hbm, idx_v, slab_v, rows_v, gsem, wsem):
        wid = lax.axis_index("s") * _NC + lax.axis_index("c")
        base = wid * _B_PER_W
        pltpu.sync_copy(idx_hbm.at[pl.ds(base, _B_PER_W)], idx_v)

        iota = lax.iota(jnp.int32, 16)

        def group(ko, _):
            gvec = idx_v[pl.ds(ko * _NBUF, _NBUF)]
            lvec_all = gvec & (LANES - 1)
            for b in range(_NBUF):
                g = gvec[b]
                off = pl.multiple_of((g >> 7) * LANES, LANES)
                pltpu.make_async_copy(
                    table_hbm.at[:, pl.ds(off, LANES)],
                    slab_v.at[b],
                    gsem,
                ).start()
            pltpu.make_async_copy(
                table_hbm.at[:, pl.ds(0, _NBUF * LANES)],
                slab_v,
                gsem,
            ).wait()
            for b in range(_NBUF):
                i = ko * _NBUF + b
                bvec = jnp.full((16,), b, jnp.int32)
                lvec = jnp.full((16,), lvec_all[b], jnp.int32)
                ivec = jnp.full((16,), i, jnp.int32)
                for k in range(D // 16):
                    rvec = iota + (16 * k)
                    vals = plsc.load_gather(slab_v, [bvec, rvec, lvec])
                    plsc.store_scatter(rows_v, [rvec, ivec], vals)
            return ()

        lax.fori_loop(0, _NGRP, group, ())
        wcp = pltpu.make_async_copy(
            rows_v, out_hbm.at[:, pl.ds(base, _B_PER_W)], wsem
        )
        wcp.start()
        wcp.wait()

    return body(indices, table_t)


def kernel(indices, sh_coeffs):
    idx = jnp.asarray(indices, jnp.int32)
    table_t = jnp.transpose(sh_coeffs, (2, 1, 0)).reshape(D, NUM_GAUSSIANS)
    out = _gather(idx, table_t)
    return jnp.transpose(out.reshape(3, 16, BATCH), (2, 1, 0))


# speedup vs baseline: 10.0208x; 1.0484x over previous
import functools

import jax
import jax.numpy as jnp
from jax import lax
from jax.experimental import pallas as pl
from jax.experimental.pallas import tpu as pltpu
from jax.experimental.pallas import tpu_sc as plsc

NUM_GAUSSIANS = 1000000
D = 48
BATCH = 16384
LANES = 128

_info = plsc.get_sparse_core_info()
_NC, _NS = _info.num_cores, _info.num_subcores
_NW = _NC * _NS
_B_PER_W = BATCH // _NW
_NBUF = 16
_NGRP = _B_PER_W // _NBUF


@jax.jit
def _gather(indices, table_t):
    mesh = plsc.VectorSubcoreMesh(core_axis_name="c", subcore_axis_name="s")

    @functools.partial(
        pl.kernel,
        out_type=jax.ShapeDtypeStruct((D, BATCH), jnp.float32),
        mesh=mesh,
        scratch_types=[
            pltpu.VMEM((_B_PER_W,), jnp.int32),
            pltpu.VMEM((_NBUF, D, LANES), jnp.float32),
            pltpu.VMEM((D, _B_PER_W), jnp.float32),
            pltpu.SemaphoreType.DMA,
            pltpu.SemaphoreType.DMA,
            pltpu.SemaphoreType.DMA,
        ],
        compiler_params=pltpu.CompilerParams(needs_layout_passes=False),
    )
    def body(idx_hbm, table_hbm, out_hbm, idx_v, slab_v, rows_v,
             sem_a, sem_b, wsem):
        wid = lax.axis_index("s") * _NC + lax.axis_index("c")
        base = wid * _B_PER_W
        pltpu.sync_copy(idx_hbm.at[pl.ds(base, _B_PER_W)], idx_v)

        iota = lax.iota(jnp.int32, 16)
        half = _NBUF // 2

        def fire(gv, lo, slot0, sem):
            for b in range(half):
                g = gv[lo + b]
                off = pl.multiple_of((g >> 7) * LANES, LANES)
                pltpu.make_async_copy(
                    table_hbm.at[:, pl.ds(off, LANES)],
                    slab_v.at[slot0 + b],
                    sem,
                ).start()

        def drain(slot0, sem):
            pltpu.make_async_copy(
                table_hbm.at[:, pl.ds(0, half * LANES)],
                slab_v.at[pl.ds(slot0, half)],
                sem,
            ).wait()

        def extract(gv, ko, lo, slot0):
            lv_all = gv & (LANES - 1)
            for b in range(half):
                i = ko * _NBUF + lo + b
                bvec = jnp.full((16,), slot0 + b, jnp.int32)
                lvec = jnp.full((16,), lv_all[lo + b], jnp.int32)
                ivec = jnp.full((16,), i, jnp.int32)
                for k in range(D // 16):
                    rvec = iota + (16 * k)
                    vals = plsc.load_gather(slab_v, [bvec, rvec, lvec])
                    plsc.store_scatter(rows_v, [rvec, ivec], vals)

        gv0 = idx_v[pl.ds(0, _NBUF)]
        fire(gv0, 0, 0, sem_a)
        fire(gv0, half, half, sem_b)

        def group(ko, _):
            gv = idx_v[pl.ds(ko * _NBUF, _NBUF)]
            drain(0, sem_a)
            extract(gv, ko, 0, 0)

            @pl.when(ko < _NGRP - 1)
            def _():
                gvn = idx_v[pl.ds((ko + 1) * _NBUF, _NBUF)]
                fire(gvn, 0, 0, sem_a)

            drain(half, sem_b)
            extract(gv, ko, half, half)

            @pl.when(ko < _NGRP - 1)
            def _():
                gvn = idx_v[pl.ds((ko + 1) * _NBUF, _NBUF)]
                fire(gvn, half, half, sem_b)

            return ()

        lax.fori_loop(0, _NGRP, group, ())
        wcp = pltpu.make_async_copy(
            rows_v, out_hbm.at[:, pl.ds(base, _B_PER_W)], wsem
        )
        wcp.start()
        wcp.wait()

    return body(indices, table_t)


def kernel(indices, sh_coeffs):
    idx = jnp.asarray(indices, jnp.int32)
    table_t = jnp.transpose(sh_coeffs, (2, 1, 0)).reshape(D, NUM_GAUSSIANS)
    out = _gather(idx, table_t)
    return jnp.transpose(out.reshape(3, 16, BATCH), (2, 1, 0))
